# baseline (speedup 1.0000x reference)
import functools

import jax
import jax.numpy as jnp
from jax import lax
from jax.experimental import pallas as pl
from jax.experimental.pallas import tpu as pltpu
from jax.experimental.pallas import tpu_sc as plsc

_N = 10000
_D = 128
_E = 320000

_CHUNK = 64
_NTILES = 32
_CPT = 160
_NCH = _NTILES * _CPT
_EPAD = _NCH * _CHUNK
_ACC_ROWS = 10112
_RPT = _ACC_ROWS // 16
_HCPT = _CPT // 4
_NBUF = 4

_sc_mesh = plsc.VectorSubcoreMesh(core_axis_name="c", subcore_axis_name="s")


@functools.partial(
    pl.kernel,
    out_type=(
        jax.ShapeDtypeStruct((_ACC_ROWS, _D), jnp.float32),
        jax.ShapeDtypeStruct((_ACC_ROWS, _D), jnp.float32),
    ),
    mesh=_sc_mesh,
    scratch_types=[
        pltpu.VMEM((_HCPT, _CHUNK), jnp.int32),
        pltpu.VMEM((_HCPT, _CHUNK), jnp.int32),
        [pltpu.VMEM((_CHUNK, _D), jnp.float32) for _ in range(_NBUF)],
        pltpu.VMEM_SHARED((_ACC_ROWS, _D), jnp.float32),
        [pltpu.SemaphoreType.DMA for _ in range(_NBUF)],
        [pltpu.SemaphoreType.DMA for _ in range(_NBUF)],
        [pltpu.SemaphoreType.DMA for _ in range(_NBUF)],
    ],
)
def _segsum(table, srcc, dstc, out0, out1, src_v, dst_v, bufs, acc, gsems, gsems2, ssems):
    cid = lax.axis_index("c")
    sid = lax.axis_index("s")
    wid = cid * 16 + sid

    def _zero_row(i, carry):
        for j in range(_D // 16):
            bufs[0][i, pl.ds(j * 16, 16)] = jnp.zeros((16,), jnp.float32)
        return carry

    with jax.named_scope("zero_acc"):
        lax.fori_loop(0, _CHUNK, _zero_row, 0)
        for k in range(_RPT // _CHUNK):
            pltpu.sync_copy(bufs[0], acc.at[pl.ds(sid * _RPT + k * _CHUNK, _CHUNK)])
        rem = _RPT % _CHUNK
        pltpu.sync_copy(
            bufs[0].at[pl.ds(0, rem)],
            acc.at[pl.ds(sid * _RPT + (_RPT // _CHUNK) * _CHUNK, rem)])
        plsc.subcore_barrier()

    def _g_start(b, j, sem):
        pltpu.make_async_copy(
            table.at[src_v.at[j, pl.ds(0, 32)]], bufs[b].at[pl.ds(0, 32)], sem).start()
        pltpu.make_async_copy(
            table.at[src_v.at[j, pl.ds(32, 32)]], bufs[b].at[pl.ds(32, 32)],
            gsems2[b]).start()

    def _g_wait(b, j, sem):
        pltpu.make_async_copy(
            table.at[src_v.at[j, pl.ds(0, 32)]], bufs[b].at[pl.ds(0, 32)], sem).wait()
        pltpu.make_async_copy(
            table.at[src_v.at[j, pl.ds(32, 32)]], bufs[b].at[pl.ds(32, 32)],
            gsems2[b]).wait()

    def _s_start(b, j, sem):
        pltpu.make_async_copy(bufs[b], acc.at[dst_v.at[j]], sem).start(add=True)

    def _s_wait(b, j, sem):
        pltpu.make_async_copy(bufs[b], acc.at[dst_v.at[j]], sem).wait()

    for h in range(_CPT // _HCPT):
      with jax.named_scope(f"edges_{h}"):
        base = wid * _CPT + h * _HCPT
        pltpu.sync_copy(srcc.at[pl.ds(base, _HCPT)], src_v)
        pltpu.sync_copy(dstc.at[pl.ds(base, _HCPT)], dst_v)
        for j in range(3):
            _g_start(j, j, gsems[j])

        def _quad(q, carry):
            for j4 in range(_NBUF):
                j = _NBUF * q + j4
                b3 = (j4 + 3) % _NBUF

                @pl.when(j >= 1)
                def _():
                    _s_wait(b3, j - 1, ssems[b3])

                _g_wait(j4, j, gsems[j4])
                _s_start(j4, j, ssems[j4])

                @pl.when(j + 3 < _HCPT)
                def _():
                    _g_start(b3, j + 3, gsems[b3])
            return carry

        lax.fori_loop(0, _HCPT // _NBUF, _quad, 0)
        _s_wait((_HCPT - 1) % _NBUF, _HCPT - 1, ssems[(_HCPT - 1) % _NBUF])
    plsc.subcore_barrier()

    with jax.named_scope("dump"):
        @pl.when(cid == 0)
        def _():
            pltpu.sync_copy(acc.at[pl.ds(sid * _RPT, _RPT)], out0.at[pl.ds(sid * _RPT, _RPT)])

        @pl.when(cid == 1)
        def _():
            pltpu.sync_copy(acc.at[pl.ds(sid * _RPT, _RPT)], out1.at[pl.ds(sid * _RPT, _RPT)])


def _mlp_body(x_ref, p0_ref, p1_ref, w1_ref, b1_ref, w2_ref, b2_ref, g_ref, be_ref, o_ref):
    h = x_ref[...] + p0_ref[...] + p1_ref[...]
    h = jnp.dot(h, w1_ref[...], preferred_element_type=jnp.float32) + b1_ref[...]
    h = jnp.maximum(h, 0.0)
    h = jnp.dot(h, w2_ref[...], preferred_element_type=jnp.float32) + b2_ref[...]
    mu = jnp.mean(h, axis=1, keepdims=True)
    c = h - mu
    var = jnp.mean(c * c, axis=1, keepdims=True)
    h = c * lax.rsqrt(var + 1e-5) * g_ref[...] + be_ref[...]
    o_ref[...] = jnp.maximum(h, 0.0)


_BLK = 1000


def _mlp(x, p0, p1, w1, b1, w2, b2, g, be):
    bs_x = pl.BlockSpec((_BLK, _D), lambda i: (i, 0))
    bs_w = pl.BlockSpec((_D, _D), lambda i: (0, 0))
    bs_v = pl.BlockSpec((1, _D), lambda i: (0, 0))
    return pl.pallas_call(
        _mlp_body,
        out_shape=jax.ShapeDtypeStruct((_N, _D), jnp.float32),
        grid=(_N // _BLK,),
        in_specs=[bs_x, bs_x, bs_x, bs_w, bs_v, bs_w, bs_v, bs_v, bs_v],
        out_specs=bs_x,
    )(x, p0, p1, w1, b1.reshape(1, _D), w2, b2.reshape(1, _D),
      g.reshape(1, _D), be.reshape(1, _D))


def _layer(h, srcc, dstc, w1, b1, w2, b2, g, be):
    p0, p1 = _segsum(h, srcc, dstc)
    return _mlp(h, p0, p1, w1, b1, w2, b2, g, be)


def kernel(x, edge_index, W1_0, b1_0, W2_0, b2_0, g_0, be_0,
           W1_1, b1_1, W2_1, b2_1, g_1, be_1):
    pad = _EPAD - _E
    pad_idx = jnp.arange(pad, dtype=jnp.int32)
    src = jnp.concatenate([edge_index[0], pad_idx % _N])
    dst = jnp.concatenate([edge_index[1], _N + pad_idx % (_ACC_ROWS - _N)])
    srcc = src.reshape(_NCH, _CHUNK)
    dstc = dst.reshape(_NCH, _CHUNK)
    h = _layer(x, srcc, dstc, W1_0, b1_0, W2_0, b2_0, g_0, be_0)
    h = _layer(h, srcc, dstc, W1_1, b1_1, W2_1, b2_1, g_1, be_1)
    return h


# speedup vs baseline: 1.0586x; 1.0586x over previous
import functools

import jax
import jax.numpy as jnp
from jax import lax
from jax.experimental import pallas as pl
from jax.experimental.pallas import tpu as pltpu
from jax.experimental.pallas import tpu_sc as plsc

_N = 10000
_D = 128
_E = 320000

_CHUNK = 64
_NCH = _E // _CHUNK
_HCPT = 40
_NQ = _NCH // _HCPT
_ACC_ROWS = 10112
_RPT = _ACC_ROWS // 16
_NBUF = 4

_sc_mesh = plsc.VectorSubcoreMesh(core_axis_name="c", subcore_axis_name="s")


@functools.partial(
    pl.kernel,
    out_type=(
        jax.ShapeDtypeStruct((_ACC_ROWS, _D), jnp.float32),
        jax.ShapeDtypeStruct((_ACC_ROWS, _D), jnp.float32),
    ),
    mesh=_sc_mesh,
    scratch_types=[
        pltpu.VMEM((_HCPT, _CHUNK), jnp.int32),
        pltpu.VMEM((_HCPT, _CHUNK), jnp.int32),
        [pltpu.VMEM((_CHUNK, _D), jnp.float32) for _ in range(_NBUF)],
        pltpu.VMEM_SHARED((_ACC_ROWS, _D), jnp.float32),
        [pltpu.SemaphoreType.DMA for _ in range(_NBUF)],
        [pltpu.SemaphoreType.DMA for _ in range(_NBUF)],
    ],
)
def _segsum(table, edges, out0, out1, src_v, dst_v, bufs, acc, gsems, ssems):
    cid = lax.axis_index("c")
    sid = lax.axis_index("s")
    wid = cid * 16 + sid

    def _g_start(b, j, sem):
        pltpu.make_async_copy(table.at[src_v.at[j]], bufs[b], sem).start()

    def _g_wait(b, j, sem):
        pltpu.make_async_copy(table.at[src_v.at[j]], bufs[b], sem).wait()

    def _s_start(b, j, sem):
        pltpu.make_async_copy(bufs[b], acc.at[dst_v.at[j]], sem).start(add=True)

    def _s_wait(b, j, sem):
        pltpu.make_async_copy(bufs[b], acc.at[dst_v.at[j]], sem).wait()

    def _load_idx(qg):
        pltpu.sync_copy(edges.at[0, pl.ds(qg * _HCPT, _HCPT)], src_v)
        pltpu.sync_copy(edges.at[1, pl.ds(qg * _HCPT, _HCPT)], dst_v)

    has_q0 = wid * 4 < _NQ
    @pl.when(has_q0)
    def _():
        _load_idx(wid * 4)
        for j in range(3):
            _g_start(j, j, gsems[j])

    def _zero_row(i, carry):
        for j in range(_D // 16):
            bufs[3][i, pl.ds(j * 16, 16)] = jnp.zeros((16,), jnp.float32)
        return carry

    with jax.named_scope("zero_acc"):
        lax.fori_loop(0, _CHUNK, _zero_row, 0)
        for k in range(_RPT // _CHUNK):
            pltpu.sync_copy(bufs[3], acc.at[pl.ds(sid * _RPT + k * _CHUNK, _CHUNK)])
        rem = _RPT % _CHUNK
        pltpu.sync_copy(
            bufs[3].at[pl.ds(0, rem)],
            acc.at[pl.ds(sid * _RPT + (_RPT // _CHUNK) * _CHUNK, rem)])
        plsc.subcore_barrier()

    def _quad(q, carry):
        for j4 in range(_NBUF):
            j = _NBUF * q + j4
            b3 = (j4 + 3) % _NBUF

            @pl.when(j >= 1)
            def _():
                _s_wait(b3, j - 1, ssems[b3])

            _g_wait(j4, j, gsems[j4])
            _s_start(j4, j, ssems[j4])

            @pl.when(j + 3 < _HCPT)
            def _():
                _g_start(b3, j + 3, gsems[b3])
        return carry

    def _run_quarter():
        lax.fori_loop(0, _HCPT // _NBUF, _quad, 0)
        _s_wait((_HCPT - 1) % _NBUF, _HCPT - 1, ssems[(_HCPT - 1) % _NBUF])

    for h in range(4):
      with jax.named_scope(f"edges_{h}"):
        qg = wid * 4 + h

        @pl.when(qg < _NQ)
        def _():
            if h > 0:
                _load_idx(qg)
                for j in range(3):
                    _g_start(j, j, gsems[j])
            _run_quarter()
    plsc.subcore_barrier()

    with jax.named_scope("dump"):
        @pl.when(cid == 0)
        def _():
            pltpu.sync_copy(acc.at[pl.ds(sid * _RPT, _RPT)], out0.at[pl.ds(sid * _RPT, _RPT)])

        @pl.when(cid == 1)
        def _():
            pltpu.sync_copy(acc.at[pl.ds(sid * _RPT, _RPT)], out1.at[pl.ds(sid * _RPT, _RPT)])


def _mlp_body(x_ref, p0_ref, p1_ref, w1_ref, b1_ref, w2_ref, b2_ref, g_ref, be_ref, o_ref):
    h = x_ref[...] + p0_ref[...] + p1_ref[...]
    h = jnp.dot(h, w1_ref[...], preferred_element_type=jnp.float32) + b1_ref[...]
    h = jnp.maximum(h, 0.0)
    h = jnp.dot(h, w2_ref[...], preferred_element_type=jnp.float32) + b2_ref[...]
    mu = jnp.mean(h, axis=1, keepdims=True)
    c = h - mu
    var = jnp.mean(c * c, axis=1, keepdims=True)
    h = c * lax.rsqrt(var + 1e-5) * g_ref[...] + be_ref[...]
    o_ref[...] = jnp.maximum(h, 0.0)


_BLK = 1000


def _mlp(x, p0, p1, w1, b1, w2, b2, g, be):
    bs_x = pl.BlockSpec((_BLK, _D), lambda i: (i, 0))
    bs_w = pl.BlockSpec((_D, _D), lambda i: (0, 0))
    bs_v = pl.BlockSpec((1, _D), lambda i: (0, 0))
    return pl.pallas_call(
        _mlp_body,
        out_shape=jax.ShapeDtypeStruct((_N, _D), jnp.float32),
        grid=(_N // _BLK,),
        in_specs=[bs_x, bs_x, bs_x, bs_w, bs_v, bs_w, bs_v, bs_v, bs_v],
        out_specs=bs_x,
    )(x, p0, p1, w1, b1.reshape(1, _D), w2, b2.reshape(1, _D),
      g.reshape(1, _D), be.reshape(1, _D))


def _layer(h, edges, w1, b1, w2, b2, g, be):
    p0, p1 = _segsum(h, edges)
    return _mlp(h, p0, p1, w1, b1, w2, b2, g, be)


def kernel(x, edge_index, W1_0, b1_0, W2_0, b2_0, g_0, be_0,
           W1_1, b1_1, W2_1, b2_1, g_1, be_1):
    edges = edge_index.reshape(2, _NCH, _CHUNK)
    h = _layer(x, edges, W1_0, b1_0, W2_0, b2_0, g_0, be_0)
    h = _layer(h, edges, W1_1, b1_1, W2_1, b2_1, g_1, be_1)
    return h
